# baseline (speedup 1.0000x reference)
import jax
import jax.numpy as jnp
from jax.experimental import pallas as pl
from jax.experimental.pallas import tpu as pltpu

B = 4
N = 4096
TN = 256
NT = N // TN
KP = 128
MS = 512
NM = N // MS


def _tc_body(t3_ref, s3T_ref, tb_ref, sb_ref, row_ref, col_ref, ca_ref, cb_ref):
    b = pl.program_id(0)
    nt = pl.program_id(1)
    tblk = t3_ref[0]
    t2 = jnp.sum(tblk * tblk, axis=1)

    @pl.when(nt == 0)
    def _():
        col_ref[...] = jnp.full((8, N), jnp.float32(jnp.inf), jnp.float32)

    rowacc = jnp.full((TN, 128), jnp.float32(jnp.inf), jnp.float32)
    for mi in range(NM):
        cref = ca_ref if mi % 2 == 0 else cb_ref
        msl = pl.ds(mi * MS, MS)
        sblk = s3T_ref[0, :, msl]
        s2 = jnp.sum(sblk * sblk, axis=0)
        cref[...] = jnp.dot(
            tb_ref[0], sb_ref[0, :, msl], preferred_element_type=jnp.float32
        )

        rowpath = s2[None, :] + cref[...]
        rp = jnp.minimum(
            jnp.minimum(rowpath[:, 0:128], rowpath[:, 128:256]),
            jnp.minimum(rowpath[:, 256:384], rowpath[:, 384:512]),
        )
        rowacc = jnp.minimum(rowacc, rp)

        colpath = t2[:, None] + cref[...]
        parts = [colpath[8 * i : 8 * i + 8, :] for i in range(TN // 8)]
        while len(parts) > 1:
            parts = [
                jnp.minimum(parts[2 * i], parts[2 * i + 1])
                for i in range(len(parts) // 2)
            ]
        col_ref[:, msl] = jnp.minimum(col_ref[:, msl], parts[0])

    rowmin = t2 + jnp.min(rowacc, axis=1)
    row_ref[pl.ds(b, 1), pl.ds(nt * TN, TN)] = rowmin.reshape(1, TN)


_tc_call = pl.pallas_call(
    _tc_body,
    grid=(B, NT),
    in_specs=[
        pl.BlockSpec((1, TN, 3), lambda b, nt: (b, nt, 0)),
        pl.BlockSpec((1, 3, N), lambda b, nt: (b, 0, 0)),
        pl.BlockSpec((1, TN, KP), lambda b, nt: (b, nt, 0)),
        pl.BlockSpec((1, KP, N), lambda b, nt: (b, 0, 0)),
    ],
    out_specs=[
        pl.BlockSpec((B, N), lambda b, nt: (0, 0)),
        pl.BlockSpec((8, N), lambda b, nt: (b, 0)),
    ],
    out_shape=[
        jax.ShapeDtypeStruct((B, N), jnp.float32),
        jax.ShapeDtypeStruct((8 * B, N), jnp.float32),
    ],
    scratch_shapes=[
        pltpu.VMEM((TN, MS), jnp.float32),
        pltpu.VMEM((TN, MS), jnp.float32),
    ],
)


def _fin_body(row_ref, col_ref, s3T_ref, o_ref):
    s2 = jnp.sum(s3T_ref[...] * s3T_ref[...], axis=1)
    colm = jnp.min(col_ref[...].reshape(B, 8, N), axis=1) + s2
    tot = jnp.sum(jnp.sqrt(jnp.maximum(row_ref[...], 0.0))) + jnp.sum(
        jnp.sqrt(jnp.maximum(colm, 0.0))
    )
    o_ref[0, 0] = tot / jnp.float32(2 * B * N)


_finish = pl.pallas_call(
    _fin_body,
    out_shape=jax.ShapeDtypeStruct((1, 1), jnp.float32),
    out_specs=pl.BlockSpec(memory_space=pltpu.SMEM),
)


def kernel(template, source):
    s3T = jnp.transpose(source, (0, 2, 1))
    tb = jnp.pad(
        template.astype(jnp.bfloat16) * jnp.bfloat16(-2.0),
        ((0, 0), (0, 0), (0, KP - 3)),
    )
    sb = jnp.transpose(
        jnp.pad(source.astype(jnp.bfloat16), ((0, 0), (0, 0), (0, KP - 3))),
        (0, 2, 1),
    )
    row, col = _tc_call(template, s3T, tb, sb)
    return row[0, 0] + col[0, 0]


# speedup vs baseline: 1.1130x; 1.1130x over previous
import jax
import jax.numpy as jnp
from jax.experimental import pallas as pl
from jax.experimental.pallas import tpu as pltpu

B = 4
N = 4096
TN = 512
NT = N // TN
KP = 128
MS = 512
NM = N // MS


def _tc_body(t3_ref, s3T_ref, tb_ref, sb_ref, row_ref, col_ref, ca_ref, cb_ref,
             racc_ref):
    b = pl.program_id(0)
    nt = pl.program_id(1)
    tblk = t3_ref[0]
    t2 = jnp.sum(tblk * tblk, axis=1)

    @pl.when(nt == 0)
    def _():
        col_ref[...] = jnp.full((8, N), jnp.float32(jnp.inf), jnp.float32)

    for mi in range(NM):
        cref = ca_ref if mi % 2 == 0 else cb_ref
        msl = pl.ds(mi * MS, MS)
        sblk = s3T_ref[0, :, msl]
        s2 = jnp.sum(sblk * sblk, axis=0)
        cref[...] = jnp.dot(
            tb_ref[0], sb_ref[0, :, msl], preferred_element_type=jnp.float32
        )

        rowpath = s2[None, :] + cref[...]
        rp = jnp.minimum(
            jnp.minimum(rowpath[:, 0:128], rowpath[:, 128:256]),
            jnp.minimum(rowpath[:, 256:384], rowpath[:, 384:512]),
        )
        if mi == 0:
            racc_ref[...] = rp
        else:
            racc_ref[...] = jnp.minimum(racc_ref[...], rp)

        colpath = t2[:, None] + cref[...]
        parts = [colpath[8 * i : 8 * i + 8, :] for i in range(TN // 8)]
        while len(parts) > 1:
            parts = [
                jnp.minimum(parts[2 * i], parts[2 * i + 1])
                for i in range(len(parts) // 2)
            ]
        col_ref[:, msl] = jnp.minimum(col_ref[:, msl], parts[0])

    rowmin = t2 + jnp.min(racc_ref[...], axis=1)
    row_ref[pl.ds(b, 1), pl.ds(nt * TN, TN)] = rowmin.reshape(1, TN)


_tc_call = pl.pallas_call(
    _tc_body,
    grid=(B, NT),
    in_specs=[
        pl.BlockSpec((1, TN, 3), lambda b, nt: (b, nt, 0)),
        pl.BlockSpec((1, 3, N), lambda b, nt: (b, 0, 0)),
        pl.BlockSpec((1, TN, KP), lambda b, nt: (b, nt, 0)),
        pl.BlockSpec((1, KP, N), lambda b, nt: (b, 0, 0)),
    ],
    out_specs=[
        pl.BlockSpec((B, N), lambda b, nt: (0, 0)),
        pl.BlockSpec((8, N), lambda b, nt: (b, 0)),
    ],
    out_shape=[
        jax.ShapeDtypeStruct((B, N), jnp.float32),
        jax.ShapeDtypeStruct((8 * B, N), jnp.float32),
    ],
    scratch_shapes=[
        pltpu.VMEM((TN, MS), jnp.float32),
        pltpu.VMEM((TN, MS), jnp.float32),
        pltpu.VMEM((TN, 128), jnp.float32),
    ],
)


def _fin_body(row_ref, col_ref, s3T_ref, o_ref):
    s2 = jnp.sum(s3T_ref[...] * s3T_ref[...], axis=1)
    colm = jnp.min(col_ref[...].reshape(B, 8, N), axis=1) + s2
    tot = jnp.sum(jnp.sqrt(jnp.maximum(row_ref[...], 0.0))) + jnp.sum(
        jnp.sqrt(jnp.maximum(colm, 0.0))
    )
    o_ref[0, 0] = tot / jnp.float32(2 * B * N)


_finish = pl.pallas_call(
    _fin_body,
    out_shape=jax.ShapeDtypeStruct((1, 1), jnp.float32),
    out_specs=pl.BlockSpec(memory_space=pltpu.SMEM),
)


def kernel(template, source):
    s3T = jnp.transpose(source, (0, 2, 1))
    tb = jnp.pad(
        template.astype(jnp.bfloat16) * jnp.bfloat16(-2.0),
        ((0, 0), (0, 0), (0, KP - 3)),
    )
    sb = jnp.transpose(
        jnp.pad(source.astype(jnp.bfloat16), ((0, 0), (0, 0), (0, KP - 3))),
        (0, 2, 1),
    )
    row, col = _tc_call(template, s3T, tb, sb)
    loss = _finish(row, col, s3T)
    return loss[0, 0]


# speedup vs baseline: 1.1188x; 1.0052x over previous
import jax
import jax.numpy as jnp
from jax.experimental import pallas as pl
from jax.experimental.pallas import tpu as pltpu

B = 4
N = 4096
TN = 1024
NT = N // TN
KP = 128
MS = 512
NM = N // MS


def _tc_body(t3_ref, s3T_ref, tb_ref, sb_ref, row_ref, col_ref, ca_ref, cb_ref,
             racc_ref):
    b = pl.program_id(0)
    nt = pl.program_id(1)
    tblk = t3_ref[0]
    t2 = jnp.sum(tblk * tblk, axis=1)

    @pl.when(nt == 0)
    def _():
        col_ref[...] = jnp.full((8, N), jnp.float32(jnp.inf), jnp.float32)

    for mi in range(NM):
        cref = ca_ref if mi % 2 == 0 else cb_ref
        msl = pl.ds(mi * MS, MS)
        sblk = s3T_ref[0, :, msl]
        s2 = jnp.sum(sblk * sblk, axis=0)
        cref[...] = jnp.dot(
            tb_ref[0], sb_ref[0, :, msl], preferred_element_type=jnp.float32
        )

        rowpath = s2[None, :] + cref[...]
        rp = jnp.minimum(
            jnp.minimum(rowpath[:, 0:128], rowpath[:, 128:256]),
            jnp.minimum(rowpath[:, 256:384], rowpath[:, 384:512]),
        )
        if mi == 0:
            racc_ref[...] = rp
        else:
            racc_ref[...] = jnp.minimum(racc_ref[...], rp)

        colpath = t2[:, None] + cref[...]
        parts = [colpath[8 * i : 8 * i + 8, :] for i in range(TN // 8)]
        while len(parts) > 1:
            parts = [
                jnp.minimum(parts[2 * i], parts[2 * i + 1])
                for i in range(len(parts) // 2)
            ]
        col_ref[:, msl] = jnp.minimum(col_ref[:, msl], parts[0])

    rowmin = t2 + jnp.min(racc_ref[...], axis=1)
    row_ref[pl.ds(b, 1), pl.ds(nt * TN, TN)] = rowmin.reshape(1, TN)


_tc_call = pl.pallas_call(
    _tc_body,
    grid=(B, NT),
    in_specs=[
        pl.BlockSpec((1, TN, 3), lambda b, nt: (b, nt, 0)),
        pl.BlockSpec((1, 3, N), lambda b, nt: (b, 0, 0)),
        pl.BlockSpec((1, TN, KP), lambda b, nt: (b, nt, 0)),
        pl.BlockSpec((1, KP, N), lambda b, nt: (b, 0, 0)),
    ],
    out_specs=[
        pl.BlockSpec((B, N), lambda b, nt: (0, 0)),
        pl.BlockSpec((8, N), lambda b, nt: (b, 0)),
    ],
    out_shape=[
        jax.ShapeDtypeStruct((B, N), jnp.float32),
        jax.ShapeDtypeStruct((8 * B, N), jnp.float32),
    ],
    scratch_shapes=[
        pltpu.VMEM((TN, MS), jnp.float32),
        pltpu.VMEM((TN, MS), jnp.float32),
        pltpu.VMEM((TN, 128), jnp.float32),
    ],
)


def _fin_body(row_ref, col_ref, s3T_ref, o_ref):
    s2 = jnp.sum(s3T_ref[...] * s3T_ref[...], axis=1)
    colm = jnp.min(col_ref[...].reshape(B, 8, N), axis=1) + s2
    tot = jnp.sum(jnp.sqrt(jnp.maximum(row_ref[...], 0.0))) + jnp.sum(
        jnp.sqrt(jnp.maximum(colm, 0.0))
    )
    o_ref[0, 0] = tot / jnp.float32(2 * B * N)


_finish = pl.pallas_call(
    _fin_body,
    out_shape=jax.ShapeDtypeStruct((1, 1), jnp.float32),
    out_specs=pl.BlockSpec(memory_space=pltpu.SMEM),
)


def kernel(template, source):
    s3T = jnp.transpose(source, (0, 2, 1))
    tb = jnp.pad(
        template.astype(jnp.bfloat16) * jnp.bfloat16(-2.0),
        ((0, 0), (0, 0), (0, KP - 3)),
    )
    sb = jnp.transpose(
        jnp.pad(source.astype(jnp.bfloat16), ((0, 0), (0, 0), (0, KP - 3))),
        (0, 2, 1),
    )
    row, col = _tc_call(template, s3T, tb, sb)
    loss = _finish(row, col, s3T)
    return loss[0, 0]
